# baseline (speedup 1.0000x reference)
import jax
import jax.numpy as jnp
from jax.experimental import pallas as pl
from jax.experimental.pallas import tpu as pltpu


def _fused_kernel(pid_ref, orient_ref, cslot_ref, cpiece_ref, corient_ref,
                  eslot_ref, epiece_ref, eorient_ref, proj_w_ref, proj_b_ref,
                  out_ref):
    row = jax.lax.broadcasted_iota(jnp.int32, (20, 1), 0)
    pid = jnp.zeros((20, 1), jnp.int32)
    oid = jnp.zeros((20, 1), jnp.int32)
    for r in range(20):
        pid = jnp.where(row == r, pid_ref[0, r], pid)
        oid = jnp.where(row == r, orient_ref[0, r], oid)

    slot_all = jnp.concatenate([cslot_ref[...], eslot_ref[...]], axis=0)
    piece_all = jnp.concatenate([cpiece_ref[...], epiece_ref[...]], axis=0)
    orient_all = jnp.concatenate([corient_ref[...], eorient_ref[...]], axis=0)

    oid_adj = oid + jnp.where(row >= 8, 3, 0)

    k20 = jax.lax.broadcasted_iota(jnp.int32, (20, 20), 1)
    onehot_p = (pid == k20).astype(jnp.float32)
    emb_piece = jnp.dot(onehot_p, piece_all,
                        preferred_element_type=jnp.float32)

    k5 = jax.lax.broadcasted_iota(jnp.int32, (20, 5), 1)
    onehot_o = (oid_adj == k5).astype(jnp.float32)
    emb_orient = jnp.dot(onehot_o, orient_all,
                         preferred_element_type=jnp.float32)

    embedded = jnp.concatenate(
        [slot_all, emb_piece, emb_orient], axis=1)

    res = (jnp.dot(embedded, proj_w_ref[...],
                   preferred_element_type=jnp.float32)
           + proj_b_ref[...].reshape(1, 256))
    out_ref[...] = res.reshape(1, 20, 256)


def kernel(piece_ids, orientations, corner_slot_w, corner_piece_w,
           corner_orient_w, edge_slot_w, edge_piece_w, edge_orient_w,
           proj_w, proj_b):
    smem = pl.BlockSpec(memory_space=pltpu.SMEM)
    vmem = pl.BlockSpec(memory_space=pltpu.VMEM)
    out = pl.pallas_call(
        _fused_kernel,
        in_specs=[smem, smem] + [vmem] * 8,
        out_shape=jax.ShapeDtypeStruct((1, 20, 256), jnp.float32),
    )(piece_ids, orientations,
      corner_slot_w, corner_piece_w, corner_orient_w,
      edge_slot_w, edge_piece_w, edge_orient_w,
      proj_w, proj_b)
    return out


# speedup vs baseline: 1.0185x; 1.0185x over previous
import jax
import jax.numpy as jnp
from jax.experimental import pallas as pl
from jax.experimental.pallas import tpu as pltpu


def _fused_kernel(pid_ref, orient_ref, cslot_ref, cpiece_ref, corient_ref,
                  eslot_ref, epiece_ref, eorient_ref, proj_w_ref, proj_b_ref,
                  out_ref):
    row = jax.lax.broadcasted_iota(jnp.int32, (20, 1), 0)
    pid = jnp.zeros((20, 1), jnp.int32)
    oid = jnp.zeros((20, 1), jnp.int32)
    for r in range(20):
        pid = jnp.where(row == r, pid_ref[0, r], pid)
        oid = jnp.where(row == r, orient_ref[0, r], oid)

    slot_all = jnp.concatenate([cslot_ref[...], eslot_ref[...]], axis=0)
    piece_all = jnp.concatenate([cpiece_ref[...], epiece_ref[...]], axis=0)
    orient_all = jnp.concatenate([corient_ref[...], eorient_ref[...]], axis=0)

    oid_adj = oid + jnp.where(row >= 8, 3, 0)

    k20 = jax.lax.broadcasted_iota(jnp.int32, (20, 20), 1)
    onehot_p = (pid == k20).astype(jnp.float32)
    emb_piece = jnp.dot(onehot_p, piece_all,
                        preferred_element_type=jnp.float32)

    k5 = jax.lax.broadcasted_iota(jnp.int32, (20, 5), 1)
    onehot_o = (oid_adj == k5).astype(jnp.float32)
    emb_orient = jnp.dot(onehot_o, orient_all,
                         preferred_element_type=jnp.float32)

    w = proj_w_ref[...]
    res = (jnp.dot(slot_all, w[0:42], preferred_element_type=jnp.float32)
           + jnp.dot(emb_piece, w[42:84], preferred_element_type=jnp.float32)
           + jnp.dot(emb_orient, w[84:128], preferred_element_type=jnp.float32)
           + proj_b_ref[...].reshape(1, 256))
    out_ref[...] = res.reshape(1, 20, 256)


def kernel(piece_ids, orientations, corner_slot_w, corner_piece_w,
           corner_orient_w, edge_slot_w, edge_piece_w, edge_orient_w,
           proj_w, proj_b):
    smem = pl.BlockSpec(memory_space=pltpu.SMEM)
    vmem = pl.BlockSpec(memory_space=pltpu.VMEM)
    out = pl.pallas_call(
        _fused_kernel,
        in_specs=[smem, smem] + [vmem] * 8,
        out_shape=jax.ShapeDtypeStruct((1, 20, 256), jnp.float32),
    )(piece_ids, orientations,
      corner_slot_w, corner_piece_w, corner_orient_w,
      edge_slot_w, edge_piece_w, edge_orient_w,
      proj_w, proj_b)
    return out


# speedup vs baseline: 1.0348x; 1.0160x over previous
import jax
import jax.numpy as jnp
from jax.experimental import pallas as pl
from jax.experimental.pallas import tpu as pltpu


def _fused_kernel(pid_ref, orient_ref, cslot_ref, cpiece_ref, corient_ref,
                  eslot_ref, epiece_ref, eorient_ref, proj_w_ref, proj_b_ref,
                  out_ref):
    row = jax.lax.broadcasted_iota(jnp.int32, (20, 1), 0)
    pid = jnp.zeros((20, 1), jnp.int32)
    oid = jnp.zeros((20, 1), jnp.int32)
    for r in range(20):
        pid = jnp.where(row == r, pid_ref[0, r], pid)
        oid = jnp.where(row == r, orient_ref[0, r], oid)

    slot_all = jnp.concatenate([cslot_ref[...], eslot_ref[...]], axis=0)
    piece_all = jnp.concatenate([cpiece_ref[...], epiece_ref[...]], axis=0)
    orient_all = jnp.concatenate([corient_ref[...], eorient_ref[...]], axis=0)

    oid_adj = oid + jnp.where(row >= 8, 3, 0)

    k20 = jax.lax.broadcasted_iota(jnp.int32, (20, 20), 1)
    onehot_p = (pid == k20).astype(jnp.float32)
    k5 = jax.lax.broadcasted_iota(jnp.int32, (20, 5), 1)
    onehot_o = (oid_adj == k5).astype(jnp.float32)

    w = proj_w_ref[...]
    pw = jnp.dot(piece_all, w[42:84],
                 preferred_element_type=jnp.float32)
    ow = jnp.dot(orient_all, w[84:128],
                 preferred_element_type=jnp.float32)
    res = (jnp.dot(slot_all, w[0:42], preferred_element_type=jnp.float32)
           + jnp.dot(onehot_p, pw, preferred_element_type=jnp.float32)
           + jnp.dot(onehot_o, ow, preferred_element_type=jnp.float32)
           + proj_b_ref[...].reshape(1, 256))
    out_ref[...] = res.reshape(1, 20, 256)


def kernel(piece_ids, orientations, corner_slot_w, corner_piece_w,
           corner_orient_w, edge_slot_w, edge_piece_w, edge_orient_w,
           proj_w, proj_b):
    smem = pl.BlockSpec(memory_space=pltpu.SMEM)
    vmem = pl.BlockSpec(memory_space=pltpu.VMEM)
    out = pl.pallas_call(
        _fused_kernel,
        in_specs=[smem, smem] + [vmem] * 8,
        out_shape=jax.ShapeDtypeStruct((1, 20, 256), jnp.float32),
    )(piece_ids, orientations,
      corner_slot_w, corner_piece_w, corner_orient_w,
      edge_slot_w, edge_piece_w, edge_orient_w,
      proj_w, proj_b)
    return out
